# baseline (speedup 1.0000x reference)
import jax
import jax.numpy as jnp
from jax import lax
from jax.experimental import pallas as pl
from jax.experimental.pallas import tpu as pltpu
from jax.experimental.pallas import tpu_sc as plsc

_N_FACTORS = 16
_BATCH = 4096

_INFO = plsc.get_sparse_core_info()
_NC = _INFO.num_cores
_NS = _INFO.num_subcores
_NW = _NC * _NS
_BPW = _BATCH // _NW
_LANES = 16


def _gather_body(users_hbm, items_hbm, ut_hbm, it_hbm, ib_hbm,
                 u_out, v_out, b_out,
                 uidx_v, iidx_v, rowidx_v, urows_v, irows_v, brows_v, bias_v,
                 sem_u, sem_v, sem_b):
    wid = lax.axis_index("s") * _NC + lax.axis_index("c")
    base = wid * _BPW
    pltpu.sync_copy(users_hbm.at[pl.ds(base, _BPW)], uidx_v)
    pltpu.sync_copy(items_hbm.at[pl.ds(base, _BPW)], iidx_v)
    for j in range(_BPW // _LANES):
        sl = pl.ds(j * _LANES, _LANES)
        rowidx_v[sl] = lax.shift_right_logical(iidx_v[sl], 4)
    cu = pltpu.async_copy(ut_hbm.at[uidx_v], urows_v, sem_u)
    cv = pltpu.async_copy(it_hbm.at[iidx_v], irows_v, sem_v)
    cb = pltpu.async_copy(ib_hbm.at[rowidx_v], brows_v, sem_b)
    cu.wait()
    pltpu.sync_copy(urows_v, u_out.at[pl.ds(base, _BPW)])
    cv.wait()
    pltpu.sync_copy(irows_v, v_out.at[pl.ds(base, _BPW)])
    cb.wait()
    for j in range(_BPW // _LANES):
        sl = pl.ds(j * _LANES, _LANES)
        lane = lax.bitwise_and(iidx_v[sl], 15)
        p = lax.iota(jnp.int32, _LANES) + j * _LANES
        bias_v[sl] = plsc.load_gather(brows_v, [p, lane])
    pltpu.sync_copy(bias_v, b_out.at[pl.ds(base, _BPW)])


_gather = pl.kernel(
    _gather_body,
    mesh=plsc.VectorSubcoreMesh(core_axis_name="c", subcore_axis_name="s"),
    out_type=[
        jax.ShapeDtypeStruct((_BATCH, _N_FACTORS), jnp.float32),
        jax.ShapeDtypeStruct((_BATCH, _N_FACTORS), jnp.float32),
        jax.ShapeDtypeStruct((_BATCH,), jnp.float32),
    ],
    scratch_types=[
        pltpu.VMEM((_BPW,), jnp.int32),
        pltpu.VMEM((_BPW,), jnp.int32),
        pltpu.VMEM((_BPW,), jnp.int32),
        pltpu.VMEM((_BPW, _N_FACTORS), jnp.float32),
        pltpu.VMEM((_BPW, _N_FACTORS), jnp.float32),
        pltpu.VMEM((_BPW, _N_FACTORS), jnp.float32),
        pltpu.VMEM((_BPW,), jnp.float32),
        pltpu.SemaphoreType.DMA,
        pltpu.SemaphoreType.DMA,
        pltpu.SemaphoreType.DMA,
    ],
    compiler_params=pltpu.CompilerParams(
        use_tc_tiling_on_sc=False, needs_layout_passes=False),
)

_BM = 512


def _mm_body(u_ref, v_ref, b_ref, o_ref):
    o_ref[...] = lax.dot_general(
        u_ref[...], v_ref[...],
        (((1,), (1,)), ((), ())),
        preferred_element_type=jnp.float32,
    ) + b_ref[...]


@jax.jit
def kernel(users, items, user_table, item_table, item_bias):
    bias_rows = item_bias.reshape(-1, _N_FACTORS)
    u, v, b = _gather(users, items, user_table, item_table, bias_rows)
    b = b.reshape(_BATCH, 1)
    return pl.pallas_call(
        _mm_body,
        grid=(_BATCH // _BM,),
        in_specs=[
            pl.BlockSpec((_BM, _N_FACTORS), lambda i: (i, 0)),
            pl.BlockSpec((_BATCH, _N_FACTORS), lambda i: (0, 0)),
            pl.BlockSpec((_BM, 1), lambda i: (i, 0)),
        ],
        out_specs=pl.BlockSpec((_BM, _BATCH), lambda i: (i, 0)),
        out_shape=jax.ShapeDtypeStruct((_BATCH, _BATCH), jnp.float32),
    )(u, v, b)


# speedup vs baseline: 6.7655x; 6.7655x over previous
import jax
import jax.numpy as jnp
from jax import lax
from jax.experimental import pallas as pl
from jax.experimental.pallas import tpu as pltpu
from jax.experimental.pallas import tpu_sc as plsc

_N_FACTORS = 16
_BATCH = 4096

_INFO = plsc.get_sparse_core_info()
_NC = _INFO.num_cores
_NS = _INFO.num_subcores
_NW = _NC * _NS
_BPW = _BATCH // _NW
_G = 16
_L = 16


def _iota16():
    return lax.iota(jnp.int32, _L)


def _splat(c):
    return jnp.full((_L,), c, jnp.int32)


def _gather_body(users_hbm, items_hbm, ut_hbm, it_hbm, ib_hbm,
                 u_out, v_out, b_out,
                 uidx_v, iidx_v,
                 ublocks, vblocks, bblocks, ucols, vcols, bvals,
                 sem_u, sem_v, sem_b):
    wid = lax.axis_index("s") * _NC + lax.axis_index("c")
    base = wid * _BPW
    pltpu.sync_copy(users_hbm.at[pl.ds(base, _BPW)], uidx_v)
    pltpu.sync_copy(items_hbm.at[pl.ds(base, _BPW)], iidx_v)
    kvec = _iota16()

    for g in range(_BPW // _G):
        sl = pl.ds(g * _G, _G)
        au_vec = lax.shift_left(lax.shift_right_logical(uidx_v[sl], 7), 7)
        ai_vec = lax.shift_left(lax.shift_right_logical(iidx_v[sl], 7), 7)
        handles = []
        for t in range(_G):
            lane_t = jnp.where(kvec == t, 1, 0)
            au = pl.multiple_of(jnp.sum(au_vec * lane_t), 128)
            ai = pl.multiple_of(jnp.sum(ai_vec * lane_t), 128)
            handles.append(pltpu.async_copy(
                ut_hbm.at[:, pl.ds(au, 128)], ublocks.at[t], sem_u))
            handles.append(pltpu.async_copy(
                it_hbm.at[:, pl.ds(ai, 128)], vblocks.at[t], sem_v))
            handles.append(pltpu.async_copy(
                ib_hbm.at[pl.ds(ai, 128)], bblocks.at[t], sem_b))
        for h in handles:
            h.wait()
        ulane = lax.bitwise_and(uidx_v[sl], 127)
        ilane = lax.bitwise_and(iidx_v[sl], 127)
        kvec = _iota16()
        outcol = kvec + g * _G
        for r in range(_N_FACTORS):
            uv = plsc.load_gather(ublocks, [kvec, _splat(r), ulane])
            plsc.store_scatter(ucols, [_splat(r), outcol], uv)
            vv = plsc.load_gather(vblocks, [kvec, _splat(r), ilane])
            plsc.store_scatter(vcols, [_splat(r), outcol], vv)
        bvals[sl] = plsc.load_gather(bblocks, [kvec, ilane])

    pltpu.sync_copy(ucols, u_out.at[:, pl.ds(base, _BPW)])
    pltpu.sync_copy(vcols, v_out.at[:, pl.ds(base, _BPW)])
    pltpu.sync_copy(bvals, b_out.at[pl.ds(base, _BPW)])


_gather = pl.kernel(
    _gather_body,
    mesh=plsc.VectorSubcoreMesh(core_axis_name="c", subcore_axis_name="s"),
    out_type=[
        jax.ShapeDtypeStruct((_N_FACTORS, _BATCH), jnp.float32),
        jax.ShapeDtypeStruct((_N_FACTORS, _BATCH), jnp.float32),
        jax.ShapeDtypeStruct((_BATCH,), jnp.float32),
    ],
    scratch_types=[
        pltpu.VMEM((_BPW,), jnp.int32),
        pltpu.VMEM((_BPW,), jnp.int32),
        pltpu.VMEM((_G, _N_FACTORS, 128), jnp.float32),
        pltpu.VMEM((_G, _N_FACTORS, 128), jnp.float32),
        pltpu.VMEM((_G, 128), jnp.float32),
        pltpu.VMEM((_N_FACTORS, _BPW), jnp.float32),
        pltpu.VMEM((_N_FACTORS, _BPW), jnp.float32),
        pltpu.VMEM((_BPW,), jnp.float32),
        pltpu.SemaphoreType.DMA,
        pltpu.SemaphoreType.DMA,
        pltpu.SemaphoreType.DMA,
    ],
    compiler_params=pltpu.CompilerParams(needs_layout_passes=False),
)

_BM = 512


def _mm_body(ut_ref, vt_ref, b_ref, o_ref):
    o_ref[...] = lax.dot_general(
        ut_ref[...], vt_ref[...],
        (((0,), (0,)), ((), ())),
        preferred_element_type=jnp.float32,
    ) + b_ref[...]


@jax.jit
def kernel(users, items, user_table, item_table, item_bias):
    u_t, v_t, b = _gather(users, items, user_table.T, item_table.T,
                          item_bias.reshape(-1))
    return pl.pallas_call(
        _mm_body,
        grid=(_BATCH // _BM,),
        in_specs=[
            pl.BlockSpec((_N_FACTORS, _BM), lambda i: (0, i)),
            pl.BlockSpec((_N_FACTORS, _BATCH), lambda i: (0, 0)),
            pl.BlockSpec((_BM, 1), lambda i: (i, 0)),
        ],
        out_specs=pl.BlockSpec((_BM, _BATCH), lambda i: (i, 0)),
        out_shape=jax.ShapeDtypeStruct((_BATCH, _BATCH), jnp.float32),
    )(u_t, v_t, b.reshape(_BATCH, 1))
